# baseline (speedup 1.0000x reference)
import functools

import jax
import jax.numpy as jnp
from jax import lax
from jax.experimental import pallas as pl
from jax.experimental.pallas import tpu as pltpu
from jax.experimental.pallas import tpu_sc as plsc

_B = 16384
_D = 64
_NC = 2
_NS = 16
_NW = _NC * _NS
_BPW = _B // _NW

_mesh = plsc.VectorSubcoreMesh(core_axis_name="c", subcore_axis_name="s")


@functools.partial(
    pl.kernel,
    out_type=(
        jax.ShapeDtypeStruct((_B,), jnp.float32),
        jax.ShapeDtypeStruct((_B, _D), jnp.float32),
        jax.ShapeDtypeStruct((_B, _D), jnp.float32),
    ),
    mesh=_mesh,
    compiler_params=pltpu.CompilerParams(
        needs_layout_passes=False, use_tc_tiling_on_sc=False
    ),
    scratch_types=[
        pltpu.VMEM((_BPW,), jnp.int32),
        pltpu.VMEM((_BPW,), jnp.int32),
        pltpu.VMEM((_BPW, _D), jnp.float32),
        pltpu.VMEM((_BPW, _D), jnp.float32),
        pltpu.VMEM((_BPW,), jnp.float32),
        pltpu.SemaphoreType.DMA,
        pltpu.SemaphoreType.DMA,
        pltpu.SemaphoreType.DMA,
        pltpu.SemaphoreType.DMA,
    ],
)
def _bprmf_sc(users_hbm, items_hbm, gu_hbm, gi_hbm,
              xui_hbm, gu_out_hbm, gi_out_hbm,
              uidx_v, iidx_v, urows_v, irows_v, xui_v,
              sem_u, sem_i, sem_ou, sem_oi):
    wid = lax.axis_index("s") * _NC + lax.axis_index("c")
    base = wid * _BPW

    pltpu.sync_copy(users_hbm.at[pl.ds(base, _BPW)], uidx_v)
    pltpu.sync_copy(items_hbm.at[pl.ds(base, _BPW)], iidx_v)
    cu = pltpu.async_copy(gu_hbm.at[uidx_v], urows_v, sem_u)
    ci = pltpu.async_copy(gi_hbm.at[iidx_v], irows_v, sem_i)
    cu.wait()
    ci.wait()

    ou = pltpu.async_copy(urows_v, gu_out_hbm.at[pl.ds(base, _BPW)], sem_ou)
    oi = pltpu.async_copy(irows_v, gi_out_hbm.at[pl.ds(base, _BPW)], sem_oi)

    rix0 = lax.iota(jnp.int32, 16)

    def group(g, carry):
        rix = rix0 + g * 16

        def col4(c4, acc):
            for dc in range(4):
                cc = jnp.broadcast_to(c4 * 4 + dc, (16,))
                u = plsc.load_gather(urows_v, [rix, cc])
                i = plsc.load_gather(irows_v, [rix, cc])
                acc = acc + u * i
            return acc

        acc = lax.fori_loop(0, _D // 4, col4, jnp.zeros((16,), jnp.float32))
        xui_v[pl.ds(g * 16, 16)] = acc
        return carry

    lax.fori_loop(0, _BPW // 16, group, 0)

    pltpu.sync_copy(xui_v, xui_hbm.at[pl.ds(base, _BPW)])
    ou.wait()
    oi.wait()


def kernel(users, items, Gu, Gi):
    users = users.astype(jnp.int32)
    items = items.astype(jnp.int32)
    xui, gamma_u, gamma_i = _bprmf_sc(users, items, Gu, Gi)
    return (xui, gamma_u, gamma_i)


# speedup vs baseline: 1.5433x; 1.5433x over previous
import functools

import jax
import jax.numpy as jnp
from jax import lax
from jax.experimental import pallas as pl
from jax.experimental.pallas import tpu as pltpu
from jax.experimental.pallas import tpu_sc as plsc

_B = 16384
_D = 64
_NC = 2
_NS = 16
_NW = _NC * _NS
_BPW = _B // _NW
_CHUNK = 256
_NCHUNK = _BPW // _CHUNK

_mesh = plsc.VectorSubcoreMesh(core_axis_name="c", subcore_axis_name="s")


@functools.partial(
    pl.kernel,
    out_type=(
        jax.ShapeDtypeStruct((_B,), jnp.float32),
        jax.ShapeDtypeStruct((_B, _D), jnp.float32),
        jax.ShapeDtypeStruct((_B, _D), jnp.float32),
    ),
    mesh=_mesh,
    compiler_params=pltpu.CompilerParams(needs_layout_passes=False),
    scratch_types=[
        pltpu.VMEM((_BPW,), jnp.int32),
        pltpu.VMEM((_BPW,), jnp.int32),
        pltpu.VMEM((_CHUNK, _D), jnp.float32),
        pltpu.VMEM((_CHUNK, _D), jnp.float32),
        pltpu.VMEM((_BPW,), jnp.float32),
        pltpu.SemaphoreType.DMA,
        pltpu.SemaphoreType.DMA,
        pltpu.SemaphoreType.DMA,
        pltpu.SemaphoreType.DMA,
    ],
)
def _bprmf_sc(users_hbm, items_hbm, gu_hbm, gi_hbm,
              xui_hbm, gu_out_hbm, gi_out_hbm,
              uidx_v, iidx_v, urows_v, irows_v, xui_v,
              sem_u, sem_i, sem_ou, sem_oi):
    wid = lax.axis_index("s") * _NC + lax.axis_index("c")
    base = wid * _BPW

    pltpu.sync_copy(users_hbm.at[pl.ds(base, _BPW)], uidx_v)
    pltpu.sync_copy(items_hbm.at[pl.ds(base, _BPW)], iidx_v)

    rix0 = lax.iota(jnp.int32, 16)

    for chunk in range(_NCHUNK):
        lo = chunk * _CHUNK

        def fire16(g, carry):
            uvec = uidx_v[pl.ds(lo + g * 16, 16)]
            ivec = iidx_v[pl.ds(lo + g * 16, 16)]
            for j in range(16):
                pltpu.async_copy(
                    gu_hbm.at[pl.ds(uvec[j], 1)],
                    urows_v.at[pl.ds(g * 16 + j, 1)], sem_u)
                pltpu.async_copy(
                    gi_hbm.at[pl.ds(ivec[j], 1)],
                    irows_v.at[pl.ds(g * 16 + j, 1)], sem_i)
            return carry

        lax.fori_loop(0, _CHUNK // 16, fire16, 0)
        pltpu.make_async_copy(
            gu_hbm.at[pl.ds(0, _CHUNK)], urows_v, sem_u).wait()
        pltpu.make_async_copy(
            gi_hbm.at[pl.ds(0, _CHUNK)], irows_v, sem_i).wait()

        def group(g, carry):
            rix = rix0 + g * 16

            def col4(c4, acc):
                for dc in range(4):
                    cc = jnp.broadcast_to(c4 * 4 + dc, (16,))
                    u = plsc.load_gather(urows_v, [rix, cc])
                    i = plsc.load_gather(irows_v, [rix, cc])
                    acc = acc + u * i
                return acc

            acc = lax.fori_loop(0, _D // 4, col4,
                                jnp.zeros((16,), jnp.float32))
            xui_v[pl.ds(lo + g * 16, 16)] = acc
            return carry

        lax.fori_loop(0, _CHUNK // 16, group, 0)

        ou = pltpu.async_copy(
            urows_v, gu_out_hbm.at[pl.ds(base + lo, _CHUNK)], sem_ou)
        oi = pltpu.async_copy(
            irows_v, gi_out_hbm.at[pl.ds(base + lo, _CHUNK)], sem_oi)
        ou.wait()
        oi.wait()

    pltpu.sync_copy(xui_v, xui_hbm.at[pl.ds(base, _BPW)])


def kernel(users, items, Gu, Gi):
    users = users.astype(jnp.int32)
    items = items.astype(jnp.int32)
    xui, gamma_u, gamma_i = _bprmf_sc(users, items, Gu, Gi)
    return (xui, gamma_u, gamma_i)
